# baseline (speedup 1.0000x reference)
import functools

import jax
import jax.numpy as jnp
from jax import lax
from jax.experimental import pallas as pl
from jax.experimental.pallas import tpu as pltpu
from jax.experimental.pallas import tpu_sc as plsc

_NC = 2
_NS = 16
_NW = _NC * _NS
_CH = 128
_G = 16


def _deg_partials(edge4, zeros16, ones16, n_pad):
    _, _, nchunk, _ = edge4.shape
    rpt = n_pad // _NS
    depth = 12
    mesh = plsc.VectorSubcoreMesh(core_axis_name="c", subcore_axis_name="s")

    @functools.partial(
        pl.kernel,
        mesh=mesh,
        out_type=(
            jax.ShapeDtypeStruct((_NC, n_pad, 16), jnp.float32),
            jax.ShapeDtypeStruct((_NC, n_pad, 16), jnp.float32),
        ),
        scratch_types=(
            pltpu.VMEM((2, nchunk, _CH), jnp.int32),
            pltpu.VMEM((_CH, 16), jnp.float32),
            pltpu.VMEM_SHARED((n_pad, 16), jnp.float32),
            pltpu.VMEM_SHARED((n_pad, 16), jnp.float32),
            pltpu.SemaphoreType.DMA,
            pltpu.SemaphoreType.DMA,
        ),
        compiler_params=pltpu.CompilerParams(use_tc_tiling_on_sc=False),
    )
    def deg_kernel(e_hbm, z_hbm, o_hbm, outs_hbm, outd_hbm,
                   idx, ones_v, accs, accd, sem_s, sem_d):
        c = lax.axis_index("c")
        s = lax.axis_index("s")
        blk = s * _NC + c
        r0 = s * rpt
        pltpu.sync_copy(z_hbm.at[pl.ds(r0, rpt)], accs.at[pl.ds(r0, rpt)])
        pltpu.sync_copy(z_hbm.at[pl.ds(r0, rpt)], accd.at[pl.ds(r0, rpt)])
        pltpu.sync_copy(o_hbm, ones_v)
        pltpu.sync_copy(e_hbm.at[:, blk], idx)
        plsc.subcore_barrier()

        for j in range(nchunk):
            pltpu.async_copy(ones_v, accs.at[idx.at[0, j]], sem_s, add=True)
            pltpu.async_copy(ones_v, accd.at[idx.at[1, j]], sem_d, add=True)
            if j >= depth:
                pltpu.make_async_copy(
                    ones_v, accs.at[idx.at[0, j - depth]], sem_s).wait()
                pltpu.make_async_copy(
                    ones_v, accd.at[idx.at[1, j - depth]], sem_d).wait()
        for j in range(max(nchunk - depth, 0), nchunk):
            pltpu.make_async_copy(
                ones_v, accs.at[idx.at[0, j]], sem_s).wait()
            pltpu.make_async_copy(
                ones_v, accd.at[idx.at[1, j]], sem_d).wait()
        plsc.subcore_barrier()
        pltpu.sync_copy(accs.at[pl.ds(r0, rpt)], outs_hbm.at[c, pl.ds(r0, rpt)])
        pltpu.sync_copy(accd.at[pl.ds(r0, rpt)], outd_hbm.at[c, pl.ds(r0, rpt)])

    return deg_kernel(edge4, zeros16, ones16)


def _seg_sum(h, edge4, zeros_w, n_pad, w, tc_tiling=True, nbuf=2, gc=_G):
    _, _, nchunk, _ = edge4.shape
    ngroups = nchunk // gc
    rpt = n_pad // _NS
    mesh = plsc.VectorSubcoreMesh(core_axis_name="c", subcore_axis_name="s")

    @functools.partial(
        pl.kernel,
        mesh=mesh,
        out_type=jax.ShapeDtypeStruct((_NC, n_pad, w), jnp.float32),
        scratch_types=(
            pltpu.VMEM((2, 2, gc, _CH), jnp.int32),
            pltpu.VMEM((nbuf, _CH, w), jnp.float32),
            pltpu.VMEM_SHARED((n_pad, w), jnp.float32),
        ) + (pltpu.SemaphoreType.DMA,) * (2 * nbuf + 1),
        compiler_params=pltpu.CompilerParams(use_tc_tiling_on_sc=tc_tiling),
    )
    def agg_kernel(h_hbm, e_hbm, z_hbm, out_hbm, idx, rows, acc, sem_i, *sems):
        c = lax.axis_index("c")
        s = lax.axis_index("s")
        blk = s * _NC + c
        r0 = s * rpt
        pltpu.sync_copy(z_hbm.at[pl.ds(r0, rpt)], acc.at[pl.ds(r0, rpt)])
        pltpu.sync_copy(e_hbm.at[:, blk, pl.ds(0, gc)], idx.at[0])
        if ngroups > 1:
            pltpu.async_copy(e_hbm.at[:, blk, pl.ds(gc, gc)], idx.at[1], sem_i)
        plsc.subcore_barrier()

        sg = sems[:nbuf]
        ss = sems[nbuf:]

        def gref(j):
            return idx.at[(j // gc) % 2, 0, j % gc]

        def sref(j):
            return idx.at[(j // gc) % 2, 1, j % gc]

        for j in range(min(nbuf - 1, nchunk)):
            pltpu.async_copy(h_hbm.at[gref(j)], rows.at[j], sg[j])
        for j in range(nchunk):
            b = j % nbuf
            pltpu.make_async_copy(h_hbm.at[gref(j)], rows.at[b], sg[b]).wait()
            pltpu.async_copy(rows.at[b], acc.at[sref(j)], ss[b], add=True)
            if j >= 1:
                bp = (j - 1) % nbuf
                pltpu.make_async_copy(
                    rows.at[bp], acc.at[sref(j - 1)], ss[bp]).wait()
            jg = j + nbuf - 1
            if jg < nchunk:
                if jg % gc == 0 and jg >= gc:
                    pltpu.make_async_copy(
                        e_hbm.at[:, blk, pl.ds((jg // gc) * gc, gc)],
                        idx.at[(jg // gc) % 2], sem_i).wait()
                bg = jg % nbuf
                pltpu.async_copy(h_hbm.at[gref(jg)], rows.at[bg], sg[bg])
            g = j // gc
            if j % gc == 1 and 1 <= g <= ngroups - 2:
                pltpu.async_copy(
                    e_hbm.at[:, blk, pl.ds((g + 1) * gc, gc)],
                    idx.at[(g + 1) % 2], sem_i)
        b_last = (nchunk - 1) % nbuf
        pltpu.make_async_copy(
            rows.at[b_last], acc.at[sref(nchunk - 1)], ss[b_last]).wait()
        plsc.subcore_barrier()
        pltpu.sync_copy(acc.at[pl.ds(r0, rpt)], out_hbm.at[c, pl.ds(r0, rpt)])

    return agg_kernel(h, edge4, zeros_w)


def _row_block(n_pad):
    r = 8
    for cand in range(8, 1025, 8):
        if n_pad % cand == 0:
            r = cand
    return r


def _mm1(x, w1, dsp):
    n_pad, d = x.shape
    h = w1.shape[1]
    r = _row_block(n_pad)

    def body(x_ref, w_ref, dsp_ref, o_ref):
        deg = dsp_ref[0, :, 0:1] + dsp_ref[1, :, 0:1]
        sc = lax.rsqrt(jnp.maximum(deg, 1.0))
        o_ref[...] = jnp.dot(x_ref[...], w_ref[...],
                             preferred_element_type=jnp.float32) * sc

    return pl.pallas_call(
        body,
        grid=(n_pad // r,),
        in_specs=[
            pl.BlockSpec((r, d), lambda i: (i, 0)),
            pl.BlockSpec((d, h), lambda i: (0, 0)),
            pl.BlockSpec((2, r, 16), lambda i: (0, i, 0)),
        ],
        out_specs=pl.BlockSpec((r, h), lambda i: (i, 0)),
        out_shape=jax.ShapeDtypeStruct((n_pad, h), jnp.float32),
    )(x, w1, dsp)


def _mm2(p1, ddp, dsp, w2p):
    _, n_pad, h = p1.shape
    wp = w2p.shape[1]
    r = _row_block(n_pad)

    def body(p1_ref, ddp_ref, dsp_ref, w_ref, o_ref):
        din = ddp_ref[0, :, 0:1] + ddp_ref[1, :, 0:1]
        dout = dsp_ref[0, :, 0:1] + dsp_ref[1, :, 0:1]
        s_in = lax.rsqrt(jnp.maximum(din, 1.0))
        s_out = lax.rsqrt(jnp.maximum(dout, 1.0))
        x2 = jnp.maximum((p1_ref[0] + p1_ref[1]) * s_in, 0.0) * s_out
        o_ref[...] = jnp.dot(x2, w_ref[...],
                             preferred_element_type=jnp.float32)

    return pl.pallas_call(
        body,
        grid=(n_pad // r,),
        in_specs=[
            pl.BlockSpec((2, r, h), lambda i: (0, i, 0)),
            pl.BlockSpec((2, r, 16), lambda i: (0, i, 0)),
            pl.BlockSpec((2, r, 16), lambda i: (0, i, 0)),
            pl.BlockSpec((h, wp), lambda i: (0, 0)),
        ],
        out_specs=pl.BlockSpec((r, wp), lambda i: (i, 0)),
        out_shape=jax.ShapeDtypeStruct((n_pad, wp), jnp.float32),
    )(p1, ddp, dsp, w2p)


def _fin(p2, ddp, c_out):
    _, n_pad, wp = p2.shape
    r = _row_block(n_pad)

    def body(p2_ref, ddp_ref, o_ref):
        din = ddp_ref[0, :, 0:1] + ddp_ref[1, :, 0:1]
        s_in = lax.rsqrt(jnp.maximum(din, 1.0))
        o_ref[...] = (p2_ref[0, :, :c_out] + p2_ref[1, :, :c_out]) * s_in

    return pl.pallas_call(
        body,
        grid=(n_pad // r,),
        in_specs=[
            pl.BlockSpec((2, r, wp), lambda i: (0, i, 0)),
            pl.BlockSpec((2, r, 16), lambda i: (0, i, 0)),
        ],
        out_specs=pl.BlockSpec((r, c_out), lambda i: (i, 0)),
        out_shape=jax.ShapeDtypeStruct((n_pad, c_out), jnp.float32),
    )(p2, ddp)


def kernel(x, edge_index, W1, W2):
    n, d = x.shape
    h = W1.shape[1]
    c = W2.shape[1]
    e = edge_index.shape[1]

    ew = -(-e // _NW)
    ew_pad = -(-ew // (_G * _CH)) * (_G * _CH)
    nchunk = ew_pad // _CH
    n_pad = -(-(n + 1) // 128) * 128

    pad_cnt = _NW * ew_pad - e
    pad_idx = n + (jnp.arange(pad_cnt, dtype=jnp.int32) % (n_pad - n))
    e2 = jnp.concatenate(
        [edge_index, jnp.broadcast_to(pad_idx, (2, pad_cnt))], axis=1)
    edge4 = e2.reshape(2, _NW, nchunk, _CH)

    wp = ((c + 15) // 16) * 16
    w2p = jnp.pad(W2, ((0, 0), (0, wp - c))) if wp != c else W2
    x_pad = jnp.pad(x, ((0, n_pad - n), (0, 0)))

    zeros16 = jnp.zeros((n_pad, 16), jnp.float32)
    ones16 = jnp.ones((_CH, 16), jnp.float32)
    zeros_h = jnp.zeros((n_pad, h), jnp.float32)
    zeros_c = jnp.zeros((n_pad, wp), jnp.float32)

    dsp, ddp = _deg_partials(edge4, zeros16, ones16, n_pad)
    h1 = _mm1(x_pad, W1, dsp)
    p1 = _seg_sum(h1, edge4, zeros_h, n_pad, h)
    h2 = _mm2(p1, ddp, dsp, w2p)
    p2 = _seg_sum(h2, edge4, zeros_c, n_pad, wp, tc_tiling=False, nbuf=4)
    return _fin(p2, ddp, c)[:n]


# speedup vs baseline: 1.0735x; 1.0735x over previous
import functools

import jax
import jax.numpy as jnp
from jax import lax
from jax.experimental import pallas as pl
from jax.experimental.pallas import tpu as pltpu
from jax.experimental.pallas import tpu_sc as plsc

_NC = 2
_NS = 16
_NW = _NC * _NS
_CH = 112
_G = 8


def _deg_partials(edge4, zeros16, ones16, n_pad):
    _, _, nchunk, _ = edge4.shape
    rpt = n_pad // _NS
    depth = 12
    mesh = plsc.VectorSubcoreMesh(core_axis_name="c", subcore_axis_name="s")

    @functools.partial(
        pl.kernel,
        mesh=mesh,
        out_type=(
            jax.ShapeDtypeStruct((_NC, n_pad, 16), jnp.float32),
            jax.ShapeDtypeStruct((_NC, n_pad, 16), jnp.float32),
        ),
        scratch_types=(
            pltpu.VMEM((2, nchunk, _CH), jnp.int32),
            pltpu.VMEM((_CH, 16), jnp.float32),
            pltpu.VMEM_SHARED((n_pad, 16), jnp.float32),
            pltpu.VMEM_SHARED((n_pad, 16), jnp.float32),
            pltpu.SemaphoreType.DMA,
            pltpu.SemaphoreType.DMA,
        ),
        compiler_params=pltpu.CompilerParams(use_tc_tiling_on_sc=False),
    )
    def deg_kernel(e_hbm, z_hbm, o_hbm, outs_hbm, outd_hbm,
                   idx, ones_v, accs, accd, sem_s, sem_d):
        c = lax.axis_index("c")
        s = lax.axis_index("s")
        blk = s * _NC + c
        r0 = s * rpt
        pltpu.sync_copy(z_hbm.at[pl.ds(r0, rpt)], accs.at[pl.ds(r0, rpt)])
        pltpu.sync_copy(z_hbm.at[pl.ds(r0, rpt)], accd.at[pl.ds(r0, rpt)])
        pltpu.sync_copy(o_hbm, ones_v)
        pltpu.sync_copy(e_hbm.at[:, blk], idx)
        plsc.subcore_barrier()

        for j in range(nchunk):
            pltpu.async_copy(ones_v, accs.at[idx.at[0, j]], sem_s, add=True)
            pltpu.async_copy(ones_v, accd.at[idx.at[1, j]], sem_d, add=True)
            if j >= depth:
                pltpu.make_async_copy(
                    ones_v, accs.at[idx.at[0, j - depth]], sem_s).wait()
                pltpu.make_async_copy(
                    ones_v, accd.at[idx.at[1, j - depth]], sem_d).wait()
        for j in range(max(nchunk - depth, 0), nchunk):
            pltpu.make_async_copy(
                ones_v, accs.at[idx.at[0, j]], sem_s).wait()
            pltpu.make_async_copy(
                ones_v, accd.at[idx.at[1, j]], sem_d).wait()
        plsc.subcore_barrier()
        pltpu.sync_copy(accs.at[pl.ds(r0, rpt)], outs_hbm.at[c, pl.ds(r0, rpt)])
        pltpu.sync_copy(accd.at[pl.ds(r0, rpt)], outd_hbm.at[c, pl.ds(r0, rpt)])

    return deg_kernel(edge4, zeros16, ones16)


def _seg_sum(h, edge4, zeros_w, n_pad, w, tc_tiling=True, nbuf=2, gc=_G):
    _, _, nchunk, _ = edge4.shape
    ngroups = nchunk // gc
    rpt = n_pad // _NS
    mesh = plsc.VectorSubcoreMesh(core_axis_name="c", subcore_axis_name="s")

    @functools.partial(
        pl.kernel,
        mesh=mesh,
        out_type=jax.ShapeDtypeStruct((_NC, n_pad, w), jnp.float32),
        scratch_types=(
            pltpu.VMEM((2, 2, gc, _CH), jnp.int32),
            pltpu.VMEM((nbuf, _CH, w), jnp.float32),
            pltpu.VMEM_SHARED((n_pad, w), jnp.float32),
        ) + (pltpu.SemaphoreType.DMA,) * (2 * nbuf + 1),
        compiler_params=pltpu.CompilerParams(use_tc_tiling_on_sc=tc_tiling),
    )
    def agg_kernel(h_hbm, e_hbm, z_hbm, out_hbm, idx, rows, acc, sem_i, *sems):
        c = lax.axis_index("c")
        s = lax.axis_index("s")
        blk = s * _NC + c
        r0 = s * rpt
        pltpu.sync_copy(z_hbm.at[pl.ds(r0, rpt)], acc.at[pl.ds(r0, rpt)])
        pltpu.sync_copy(e_hbm.at[:, blk, pl.ds(0, gc)], idx.at[0])
        if ngroups > 1:
            pltpu.async_copy(e_hbm.at[:, blk, pl.ds(gc, gc)], idx.at[1], sem_i)
        plsc.subcore_barrier()

        sg = sems[:nbuf]
        ss = sems[nbuf:]

        def gref(j):
            return idx.at[(j // gc) % 2, 0, j % gc]

        def sref(j):
            return idx.at[(j // gc) % 2, 1, j % gc]

        for j in range(min(nbuf - 1, nchunk)):
            pltpu.async_copy(h_hbm.at[gref(j)], rows.at[j], sg[j])
        for j in range(nchunk):
            b = j % nbuf
            pltpu.make_async_copy(h_hbm.at[gref(j)], rows.at[b], sg[b]).wait()
            pltpu.async_copy(rows.at[b], acc.at[sref(j)], ss[b], add=True)
            if j >= 1:
                bp = (j - 1) % nbuf
                pltpu.make_async_copy(
                    rows.at[bp], acc.at[sref(j - 1)], ss[bp]).wait()
            jg = j + nbuf - 1
            if jg < nchunk:
                if jg % gc == 0 and jg >= gc:
                    pltpu.make_async_copy(
                        e_hbm.at[:, blk, pl.ds((jg // gc) * gc, gc)],
                        idx.at[(jg // gc) % 2], sem_i).wait()
                bg = jg % nbuf
                pltpu.async_copy(h_hbm.at[gref(jg)], rows.at[bg], sg[bg])
            g = j // gc
            if j % gc == 1 and 1 <= g <= ngroups - 2:
                pltpu.async_copy(
                    e_hbm.at[:, blk, pl.ds((g + 1) * gc, gc)],
                    idx.at[(g + 1) % 2], sem_i)
        b_last = (nchunk - 1) % nbuf
        pltpu.make_async_copy(
            rows.at[b_last], acc.at[sref(nchunk - 1)], ss[b_last]).wait()
        plsc.subcore_barrier()
        pltpu.sync_copy(acc.at[pl.ds(r0, rpt)], out_hbm.at[c, pl.ds(r0, rpt)])

    return agg_kernel(h, edge4, zeros_w)


def _row_block(n_pad):
    r = 8
    for cand in range(8, 1025, 8):
        if n_pad % cand == 0:
            r = cand
    return r


def _mm1(x, w1, dsp):
    n_pad, d = x.shape
    h = w1.shape[1]
    r = _row_block(n_pad)

    def body(x_ref, w_ref, dsp_ref, o_ref):
        deg = dsp_ref[0, :, 0:1] + dsp_ref[1, :, 0:1]
        sc = lax.rsqrt(jnp.maximum(deg, 1.0))
        o_ref[...] = jnp.dot(x_ref[...], w_ref[...],
                             preferred_element_type=jnp.float32) * sc

    return pl.pallas_call(
        body,
        grid=(n_pad // r,),
        in_specs=[
            pl.BlockSpec((r, d), lambda i: (i, 0)),
            pl.BlockSpec((d, h), lambda i: (0, 0)),
            pl.BlockSpec((2, r, 16), lambda i: (0, i, 0)),
        ],
        out_specs=pl.BlockSpec((r, h), lambda i: (i, 0)),
        out_shape=jax.ShapeDtypeStruct((n_pad, h), jnp.float32),
    )(x, w1, dsp)


def _mm2(p1, ddp, dsp, w2p):
    _, n_pad, h = p1.shape
    wp = w2p.shape[1]
    r = _row_block(n_pad)

    def body(p1_ref, ddp_ref, dsp_ref, w_ref, o_ref):
        din = ddp_ref[0, :, 0:1] + ddp_ref[1, :, 0:1]
        dout = dsp_ref[0, :, 0:1] + dsp_ref[1, :, 0:1]
        s_in = lax.rsqrt(jnp.maximum(din, 1.0))
        s_out = lax.rsqrt(jnp.maximum(dout, 1.0))
        x2 = jnp.maximum((p1_ref[0] + p1_ref[1]) * s_in, 0.0) * s_out
        o_ref[...] = jnp.dot(x2, w_ref[...],
                             preferred_element_type=jnp.float32)

    return pl.pallas_call(
        body,
        grid=(n_pad // r,),
        in_specs=[
            pl.BlockSpec((2, r, h), lambda i: (0, i, 0)),
            pl.BlockSpec((2, r, 16), lambda i: (0, i, 0)),
            pl.BlockSpec((2, r, 16), lambda i: (0, i, 0)),
            pl.BlockSpec((h, wp), lambda i: (0, 0)),
        ],
        out_specs=pl.BlockSpec((r, wp), lambda i: (i, 0)),
        out_shape=jax.ShapeDtypeStruct((n_pad, wp), jnp.float32),
    )(p1, ddp, dsp, w2p)


def _fin(p2, ddp, c_out):
    _, n_pad, wp = p2.shape
    r = _row_block(n_pad)

    def body(p2_ref, ddp_ref, o_ref):
        din = ddp_ref[0, :, 0:1] + ddp_ref[1, :, 0:1]
        s_in = lax.rsqrt(jnp.maximum(din, 1.0))
        o_ref[...] = (p2_ref[0, :, :c_out] + p2_ref[1, :, :c_out]) * s_in

    return pl.pallas_call(
        body,
        grid=(n_pad // r,),
        in_specs=[
            pl.BlockSpec((2, r, wp), lambda i: (0, i, 0)),
            pl.BlockSpec((2, r, 16), lambda i: (0, i, 0)),
        ],
        out_specs=pl.BlockSpec((r, c_out), lambda i: (i, 0)),
        out_shape=jax.ShapeDtypeStruct((n_pad, c_out), jnp.float32),
    )(p2, ddp)


def kernel(x, edge_index, W1, W2):
    n, d = x.shape
    h = W1.shape[1]
    c = W2.shape[1]
    e = edge_index.shape[1]

    ew = -(-e // _NW)
    ew_pad = -(-ew // (_G * _CH)) * (_G * _CH)
    nchunk = ew_pad // _CH
    n_pad = -(-(n + 1) // 128) * 128

    pad_cnt = _NW * ew_pad - e
    pad_idx = n + (jnp.arange(pad_cnt, dtype=jnp.int32) % (n_pad - n))
    e2 = jnp.concatenate(
        [edge_index, jnp.broadcast_to(pad_idx, (2, pad_cnt))], axis=1)
    edge4 = e2.reshape(2, _NW, nchunk, _CH)

    wp = ((c + 15) // 16) * 16
    w2p = jnp.pad(W2, ((0, 0), (0, wp - c))) if wp != c else W2
    x_pad = jnp.pad(x, ((0, n_pad - n), (0, 0)))

    zeros16 = jnp.zeros((n_pad, 16), jnp.float32)
    ones16 = jnp.ones((_CH, 16), jnp.float32)
    zeros_h = jnp.zeros((n_pad, h), jnp.float32)
    zeros_c = jnp.zeros((n_pad, wp), jnp.float32)

    dsp, ddp = _deg_partials(edge4, zeros16, ones16, n_pad)
    h1 = _mm1(x_pad, W1, dsp)
    p1 = _seg_sum(h1, edge4, zeros_h, n_pad, h, nbuf=3)
    h2 = _mm2(p1, ddp, dsp, w2p)
    p2 = _seg_sum(h2, edge4, zeros_c, n_pad, wp, tc_tiling=False, nbuf=4)
    return _fin(p2, ddp, c)[:n]


# speedup vs baseline: 1.0762x; 1.0026x over previous
import functools

import jax
import jax.numpy as jnp
from jax import lax
from jax.experimental import pallas as pl
from jax.experimental.pallas import tpu as pltpu
from jax.experimental.pallas import tpu_sc as plsc

_NC = 2
_NS = 16
_NW = _NC * _NS
_CH = 112
_G = 8


def _deg_partials(edge4, zeros16, ones16, n_pad):
    _, _, nchunk, _ = edge4.shape
    rpt = n_pad // _NS
    depth = 12
    mesh = plsc.VectorSubcoreMesh(core_axis_name="c", subcore_axis_name="s")

    @functools.partial(
        pl.kernel,
        mesh=mesh,
        out_type=(
            jax.ShapeDtypeStruct((_NC, n_pad, 16), jnp.float32),
            jax.ShapeDtypeStruct((_NC, n_pad, 16), jnp.float32),
        ),
        scratch_types=(
            pltpu.VMEM((2, nchunk, _CH), jnp.int32),
            pltpu.VMEM((_CH, 16), jnp.float32),
            pltpu.VMEM_SHARED((n_pad, 16), jnp.float32),
            pltpu.VMEM_SHARED((n_pad, 16), jnp.float32),
            pltpu.SemaphoreType.DMA,
            pltpu.SemaphoreType.DMA,
        ),
        compiler_params=pltpu.CompilerParams(use_tc_tiling_on_sc=False),
    )
    def deg_kernel(e_hbm, z_hbm, o_hbm, outs_hbm, outd_hbm,
                   idx, ones_v, accs, accd, sem_s, sem_d):
        c = lax.axis_index("c")
        s = lax.axis_index("s")
        blk = s * _NC + c
        r0 = s * rpt
        pltpu.sync_copy(z_hbm.at[pl.ds(r0, rpt)], accs.at[pl.ds(r0, rpt)])
        pltpu.sync_copy(z_hbm.at[pl.ds(r0, rpt)], accd.at[pl.ds(r0, rpt)])
        pltpu.sync_copy(o_hbm, ones_v)
        pltpu.sync_copy(e_hbm.at[:, blk], idx)
        plsc.subcore_barrier()

        for j in range(nchunk):
            pltpu.async_copy(ones_v, accs.at[idx.at[0, j]], sem_s, add=True)
            pltpu.async_copy(ones_v, accd.at[idx.at[1, j]], sem_d, add=True)
            if j >= depth:
                pltpu.make_async_copy(
                    ones_v, accs.at[idx.at[0, j - depth]], sem_s).wait()
                pltpu.make_async_copy(
                    ones_v, accd.at[idx.at[1, j - depth]], sem_d).wait()
        for j in range(max(nchunk - depth, 0), nchunk):
            pltpu.make_async_copy(
                ones_v, accs.at[idx.at[0, j]], sem_s).wait()
            pltpu.make_async_copy(
                ones_v, accd.at[idx.at[1, j]], sem_d).wait()
        plsc.subcore_barrier()
        pltpu.sync_copy(accs.at[pl.ds(r0, rpt)], outs_hbm.at[c, pl.ds(r0, rpt)])
        pltpu.sync_copy(accd.at[pl.ds(r0, rpt)], outd_hbm.at[c, pl.ds(r0, rpt)])

    return deg_kernel(edge4, zeros16, ones16)


def _seg_sum(h, edge4, zeros_w, n_pad, w, tc_tiling=True, nbuf=2, gc=_G):
    _, _, nchunk, _ = edge4.shape
    ngroups = nchunk // gc
    rpt = n_pad // _NS
    mesh = plsc.VectorSubcoreMesh(core_axis_name="c", subcore_axis_name="s")

    @functools.partial(
        pl.kernel,
        mesh=mesh,
        out_type=jax.ShapeDtypeStruct((_NC, n_pad, w), jnp.float32),
        scratch_types=(
            pltpu.VMEM((2, 2, gc, _CH), jnp.int32),
            pltpu.VMEM((nbuf, _CH, w), jnp.float32),
            pltpu.VMEM_SHARED((n_pad, w), jnp.float32),
        ) + (pltpu.SemaphoreType.DMA,) * (2 * nbuf + 1),
        compiler_params=pltpu.CompilerParams(use_tc_tiling_on_sc=tc_tiling),
    )
    def agg_kernel(h_hbm, e_hbm, z_hbm, out_hbm, idx, rows, acc, sem_i, *sems):
        c = lax.axis_index("c")
        s = lax.axis_index("s")
        blk = s * _NC + c
        r0 = s * rpt
        pltpu.sync_copy(z_hbm.at[pl.ds(r0, rpt)], acc.at[pl.ds(r0, rpt)])
        pltpu.sync_copy(e_hbm.at[:, blk, pl.ds(0, gc)], idx.at[0])
        if ngroups > 1:
            pltpu.async_copy(e_hbm.at[:, blk, pl.ds(gc, gc)], idx.at[1], sem_i)
        plsc.subcore_barrier()

        sg = sems[:nbuf]
        ss = sems[nbuf:]

        def gref(j):
            return idx.at[(j // gc) % 2, 0, j % gc]

        def sref(j):
            return idx.at[(j // gc) % 2, 1, j % gc]

        for j in range(min(nbuf - 1, nchunk)):
            pltpu.async_copy(h_hbm.at[gref(j)], rows.at[j], sg[j])
        for j in range(nchunk):
            b = j % nbuf
            pltpu.make_async_copy(h_hbm.at[gref(j)], rows.at[b], sg[b]).wait()
            pltpu.async_copy(rows.at[b], acc.at[sref(j)], ss[b], add=True)
            if j >= 1:
                bp = (j - 1) % nbuf
                pltpu.make_async_copy(
                    rows.at[bp], acc.at[sref(j - 1)], ss[bp]).wait()
            jg = j + nbuf - 1
            if jg < nchunk:
                if jg % gc == 0 and jg >= gc:
                    pltpu.make_async_copy(
                        e_hbm.at[:, blk, pl.ds((jg // gc) * gc, gc)],
                        idx.at[(jg // gc) % 2], sem_i).wait()
                bg = jg % nbuf
                pltpu.async_copy(h_hbm.at[gref(jg)], rows.at[bg], sg[bg])
            g = j // gc
            if j % gc == 1 and 1 <= g <= ngroups - 2:
                pltpu.async_copy(
                    e_hbm.at[:, blk, pl.ds((g + 1) * gc, gc)],
                    idx.at[(g + 1) % 2], sem_i)
        b_last = (nchunk - 1) % nbuf
        pltpu.make_async_copy(
            rows.at[b_last], acc.at[sref(nchunk - 1)], ss[b_last]).wait()
        plsc.subcore_barrier()
        pltpu.sync_copy(acc.at[pl.ds(r0, rpt)], out_hbm.at[c, pl.ds(r0, rpt)])

    return agg_kernel(h, edge4, zeros_w)


def _row_block(n_pad):
    r = 8
    for cand in range(8, 1025, 8):
        if n_pad % cand == 0:
            r = cand
    return r


def _mm1(x, w1, dsp):
    n_pad, d = x.shape
    h = w1.shape[1]
    r = _row_block(n_pad)

    def body(x_ref, w_ref, dsp_ref, o_ref):
        deg = dsp_ref[0, :, 0:1] + dsp_ref[1, :, 0:1]
        sc = lax.rsqrt(jnp.maximum(deg, 1.0))
        o_ref[...] = jnp.dot(x_ref[...], w_ref[...],
                             preferred_element_type=jnp.float32) * sc

    return pl.pallas_call(
        body,
        grid=(n_pad // r,),
        in_specs=[
            pl.BlockSpec((r, d), lambda i: (i, 0)),
            pl.BlockSpec((d, h), lambda i: (0, 0)),
            pl.BlockSpec((2, r, 16), lambda i: (0, i, 0)),
        ],
        out_specs=pl.BlockSpec((r, h), lambda i: (i, 0)),
        out_shape=jax.ShapeDtypeStruct((n_pad, h), jnp.float32),
    )(x, w1, dsp)


def _mm2(p1, ddp, dsp, w2p):
    _, n_pad, h = p1.shape
    wp = w2p.shape[1]
    r = _row_block(n_pad)

    def body(p1_ref, ddp_ref, dsp_ref, w_ref, o_ref):
        din = ddp_ref[0, :, 0:1] + ddp_ref[1, :, 0:1]
        dout = dsp_ref[0, :, 0:1] + dsp_ref[1, :, 0:1]
        s_in = lax.rsqrt(jnp.maximum(din, 1.0))
        s_out = lax.rsqrt(jnp.maximum(dout, 1.0))
        x2 = jnp.maximum((p1_ref[0] + p1_ref[1]) * s_in, 0.0) * s_out
        o_ref[...] = jnp.dot(x2, w_ref[...],
                             preferred_element_type=jnp.float32)

    return pl.pallas_call(
        body,
        grid=(n_pad // r,),
        in_specs=[
            pl.BlockSpec((2, r, h), lambda i: (0, i, 0)),
            pl.BlockSpec((2, r, 16), lambda i: (0, i, 0)),
            pl.BlockSpec((2, r, 16), lambda i: (0, i, 0)),
            pl.BlockSpec((h, wp), lambda i: (0, 0)),
        ],
        out_specs=pl.BlockSpec((r, wp), lambda i: (i, 0)),
        out_shape=jax.ShapeDtypeStruct((n_pad, wp), jnp.float32),
    )(p1, ddp, dsp, w2p)


def _fin(p2, ddp, c_out):
    _, n_pad, wp = p2.shape
    r = _row_block(n_pad)

    def body(p2_ref, ddp_ref, o_ref):
        din = ddp_ref[0, :, 0:1] + ddp_ref[1, :, 0:1]
        s_in = lax.rsqrt(jnp.maximum(din, 1.0))
        o_ref[...] = (p2_ref[0, :, :c_out] + p2_ref[1, :, :c_out]) * s_in

    return pl.pallas_call(
        body,
        grid=(n_pad // r,),
        in_specs=[
            pl.BlockSpec((2, r, wp), lambda i: (0, i, 0)),
            pl.BlockSpec((2, r, 16), lambda i: (0, i, 0)),
        ],
        out_specs=pl.BlockSpec((r, c_out), lambda i: (i, 0)),
        out_shape=jax.ShapeDtypeStruct((n_pad, c_out), jnp.float32),
    )(p2, ddp)


def kernel(x, edge_index, W1, W2):
    n, d = x.shape
    h = W1.shape[1]
    c = W2.shape[1]
    e = edge_index.shape[1]

    ew = -(-e // _NW)
    ew_pad = -(-ew // (_G * _CH)) * (_G * _CH)
    nchunk = ew_pad // _CH
    n_pad = -(-(n + 1) // 128) * 128

    pad_cnt = _NW * ew_pad - e
    pad_idx = n + (jnp.arange(pad_cnt, dtype=jnp.int32) % (n_pad - n))
    e2 = jnp.concatenate(
        [edge_index, jnp.broadcast_to(pad_idx, (2, pad_cnt))], axis=1)
    edge4 = e2.reshape(2, _NW, nchunk, _CH)

    wp = ((c + 15) // 16) * 16
    w2p = jnp.pad(W2, ((0, 0), (0, wp - c))) if wp != c else W2
    x_pad = jnp.pad(x, ((0, n_pad - n), (0, 0)))

    zeros16 = jnp.zeros((n_pad, 16), jnp.float32)
    ones16 = jnp.ones((_CH, 16), jnp.float32)
    zeros_h = jnp.zeros((n_pad, h), jnp.float32)
    zeros_c = jnp.zeros((n_pad, wp), jnp.float32)

    dsp, ddp = _deg_partials(edge4, zeros16, ones16, n_pad)
    h1 = _mm1(x_pad, W1, dsp)
    p1 = _seg_sum(h1, edge4, zeros_h, n_pad, h, nbuf=3)
    h2 = _mm2(p1, ddp, dsp, w2p)
    p2 = _seg_sum(h2, edge4, zeros_c, n_pad, wp, tc_tiling=False, nbuf=6)
    return _fin(p2, ddp, c)[:n]
